# Initial kernel scaffold; baseline (speedup 1.0000x reference)
import jax
import jax.numpy as jnp
from jax.experimental import pallas as pl


def kernel(x, table):
    raise NotImplementedError("write your pallas kernel here")


# speedup vs baseline: 1.4842x; 1.4842x over previous
import functools

import jax
import jax.numpy as jnp
from jax import lax
from jax.experimental import pallas as pl
from jax.experimental.pallas import tpu as pltpu
from jax.experimental.pallas import tpu_sc as plsc

_BATCH, _HIST, _DIM = 16384, 20, 32
_B = _BATCH * _HIST
_NC, _NS = 2, 16
_NW = _NC * _NS
_BPW = _B // _NW
_CHUNK = 1024
_NCHUNK = _BPW // _CHUNK
_IPD = 128
_NDMA = _CHUNK // _IPD

_mesh = plsc.VectorSubcoreMesh(core_axis_name="c", subcore_axis_name="s")


@functools.partial(
    pl.kernel,
    mesh=_mesh,
    out_type=jax.ShapeDtypeStruct((_B, _DIM), jnp.float32),
    scratch_types=[
        pltpu.VMEM((_NDMA, _IPD), jnp.int32),
        pltpu.VMEM((_CHUNK, _DIM), jnp.float32),
        pltpu.SemaphoreType.DMA,
    ],
    compiler_params=pltpu.CompilerParams(use_tc_tiling_on_sc=False),
)
def _emb_gather(x_hbm, table_hbm, out_hbm, idx_v, rows_v, sem):
    wid = lax.axis_index("s") * _NC + lax.axis_index("c")
    row0 = wid * (_BPW // _IPD)

    def chunk_body(ci, carry):
        r = row0 + ci * _NDMA
        pltpu.sync_copy(x_hbm.at[pl.ds(r, _NDMA)], idx_v)

        def dbl(t, c2):
            j = t // (_IPD // 16)
            i = lax.rem(t, _IPD // 16)
            sl = pl.ds(i * 16, 16)
            v = idx_v[j, sl]
            idx_v[j, sl] = v + v
            return c2

        lax.fori_loop(0, _CHUNK // 16, dbl, 0)

        copies = [
            pltpu.async_copy(
                table_hbm.at[idx_v.at[j]],
                rows_v.at[pl.ds(j * _IPD, _IPD)],
                sem,
            )
            for j in range(_NDMA)
        ]
        for c in copies:
            c.wait()

        off = wid * _BPW + ci * _CHUNK
        pltpu.sync_copy(rows_v, out_hbm.at[pl.ds(off, _CHUNK)])
        return carry

    lax.fori_loop(0, _NCHUNK, chunk_body, 0)


def kernel(x, table):
    x2d = x.reshape(_B // _IPD, _IPD).astype(jnp.int32)
    out = _emb_gather(x2d, table)
    return out.reshape(_BATCH, _HIST, _DIM)


# speedup vs baseline: 1.5039x; 1.0133x over previous
import functools

import jax
import jax.numpy as jnp
from jax import lax
from jax.experimental import pallas as pl
from jax.experimental.pallas import tpu as pltpu
from jax.experimental.pallas import tpu_sc as plsc

_BATCH, _HIST, _DIM = 16384, 20, 32
_B = _BATCH * _HIST
_NC, _NS = 2, 16
_NW = _NC * _NS
_BPW = _B // _NW
_CHUNK = 1024
_NCHUNK = _BPW // _CHUNK
_IPD = 128
_NDMA = _CHUNK // _IPD
_NBUF = 3

_mesh = plsc.VectorSubcoreMesh(core_axis_name="c", subcore_axis_name="s")


@functools.partial(
    pl.kernel,
    mesh=_mesh,
    out_type=jax.ShapeDtypeStruct((_B, _DIM), jnp.float32),
    scratch_types=[
        pltpu.VMEM((_BPW // _IPD, _IPD), jnp.int32),
        pltpu.VMEM((_NBUF, _CHUNK, _DIM), jnp.float32),
        pltpu.SemaphoreType.DMA((_NBUF,)),
        pltpu.SemaphoreType.DMA((_NBUF,)),
    ],
    compiler_params=pltpu.CompilerParams(use_tc_tiling_on_sc=False),
)
def _emb_gather(x_hbm, table_hbm, out_hbm, idx_v, rows_v, gsem, wsem):
    wid = lax.axis_index("s") * _NC + lax.axis_index("c")
    row0 = wid * (_BPW // _IPD)

    pltpu.sync_copy(x_hbm.at[pl.ds(row0, _BPW // _IPD)], idx_v)

    def dbl(t, c2):
        j = t // (_IPD // 16)
        i = lax.rem(t, _IPD // 16)
        sl = pl.ds(i * 16, 16)
        v = idx_v[j, sl]
        idx_v[j, sl] = v + v
        return c2

    lax.fori_loop(0, _BPW // 16, dbl, 0)

    def issue_gather(ci):
        b = ci % _NBUF
        return [
            pltpu.async_copy(
                table_hbm.at[idx_v.at[ci * _NDMA + j]],
                rows_v.at[b, pl.ds(j * _IPD, _IPD)],
                gsem.at[b],
            )
            for j in range(_NDMA)
        ]

    gathers = [issue_gather(0), issue_gather(1)] + [None] * (_NCHUNK - 2)
    writes = [None] * _NCHUNK
    for ci in range(_NCHUNK):
        b = ci % _NBUF
        for c in gathers[ci]:
            c.wait()
        writes[ci] = pltpu.async_copy(
            rows_v.at[b],
            out_hbm.at[pl.ds(wid * _BPW + ci * _CHUNK, _CHUNK)],
            wsem.at[b],
        )
        nxt = ci + 2
        if nxt < _NCHUNK:
            if writes[nxt - _NBUF] is not None:
                writes[nxt - _NBUF].wait()
            gathers[nxt] = issue_gather(nxt)
    for ci in range(_NCHUNK - _NBUF, _NCHUNK):
        writes[ci].wait()


def kernel(x, table):
    x2d = x.reshape(_B // _IPD, _IPD).astype(jnp.int32)
    out = _emb_gather(x2d, table)
    return out.reshape(_BATCH, _HIST, _DIM)


# speedup vs baseline: 1.6312x; 1.0847x over previous
import functools

import jax
import jax.numpy as jnp
import numpy as np
from jax import lax
from jax.experimental import pallas as pl
from jax.experimental.pallas import tpu as pltpu
from jax.experimental.pallas import tpu_sc as plsc

_BATCH, _HIST, _DIM = 16384, 20, 32
_NE = 1000000
_NC, _NS = 2, 16
_NW = _NC * _NS
_NCOL = _NE // 128
_CPW = 244
_NJ = 125000

_BLK = 256
_NTASK = _HIST * (_BATCH // _BLK)
_TPW = _NTASK // _NW

_mesh = plsc.VectorSubcoreMesh(core_axis_name="c", subcore_axis_name="s")


def _wid():
    return lax.axis_index("s") * _NC + lax.axis_index("c")


def _i16():
    return lax.iota(jnp.int32, 16)


@functools.partial(
    pl.kernel,
    mesh=_mesh,
    out_type=jax.ShapeDtypeStruct((_NJ, 128), jnp.float32),
    scratch_types=[
        pltpu.VMEM((32, 128), jnp.float32),
        pltpu.VMEM((32, 128), jnp.float32),
        pltpu.VMEM((16, 128), jnp.float32),
        pltpu.VMEM((16, 128), jnp.float32),
        pltpu.VMEM((32, 64), jnp.float32),
        pltpu.SemaphoreType.DMA,
        pltpu.SemaphoreType.DMA,
        pltpu.SemaphoreType.DMA,
        pltpu.SemaphoreType.DMA,
    ],
    compiler_params=pltpu.CompilerParams(needs_layout_passes=False),
)
def _detrans(tt_hbm, te_hbm, vin0, vin1, vout0, vout1, vtail,
             isem0, isem1, osem0, osem1):
    wid = _wid()
    start = wid * _CPW
    it = _i16()
    src = [(it << 1) + (32 * g) for g in range(4)]
    jl = [(it >> 2) + (4 * g) for g in range(4)]
    dlb = (it & 3) << 5
    dl = [dlb + d for d in range(32)]

    def transpose_col(vin, vout):
        for d in range(32):
            dv = jnp.full((16,), d, jnp.int32)
            for g in range(4):
                vals = plsc.load_gather(vin, [dv, src[g]])
                plsc.store_scatter(vout, [jl[g], dl[d]], vals)

    def col_slice(c):
        return tt_hbm.at[:, pl.ds(c * 128, 128)]

    def out_slice(c):
        return te_hbm.at[pl.ds(c * 16, 16)]

    pltpu.async_copy(col_slice(start), vin0, isem0)
    pltpu.async_copy(col_slice(start + 1), vin1, isem1)

    def pair(i2, carry):
        ca = start + 2 * i2
        cb = ca + 1
        for (c, vin, isem, vout, osem, nxt) in (
            (ca, vin0, isem0, vout0, osem0, ca + 2),
            (cb, vin1, isem1, vout1, osem1, cb + 2),
        ):
            pltpu.make_async_copy(col_slice(c), vin, isem).wait()

            @pl.when(i2 > 0)
            def _():
                pltpu.make_async_copy(te_hbm.at[pl.ds(0, 16)], vout,
                                      osem).wait()

            transpose_col(vin, vout)
            pltpu.async_copy(vout, out_slice(c), osem)
            cn = jnp.minimum(nxt, start + _CPW - 1 + 2)
            cn = jnp.minimum(cn, _NCOL - 1)
            pltpu.async_copy(col_slice(cn), vin, isem)
        return carry

    lax.fori_loop(0, _CPW // 2, pair, 0)
    pltpu.make_async_copy(col_slice(0), vin0, isem0).wait()
    pltpu.make_async_copy(col_slice(0), vin1, isem1).wait()
    pltpu.make_async_copy(te_hbm.at[pl.ds(0, 16)], vout0, osem0).wait()
    pltpu.make_async_copy(te_hbm.at[pl.ds(0, 16)], vout1, osem1).wait()

    @pl.when(wid < 4)
    def _():
        c = _CPW * _NW + wid
        pltpu.async_copy(col_slice(c), vin0, isem0)
        pltpu.make_async_copy(col_slice(c), vin0, isem0).wait()
        transpose_col(vin0, vout0)
        pltpu.async_copy(vout0, out_slice(c), osem0)
        pltpu.make_async_copy(te_hbm.at[pl.ds(0, 16)], vout0, osem0).wait()

    @pl.when(wid == 31)
    def _():
        pltpu.async_copy(tt_hbm.at[:, pl.ds(_NCOL * 128, 64)], vtail, isem0)
        pltpu.make_async_copy(tt_hbm.at[:, pl.ds(_NCOL * 128, 64)], vtail,
                              isem0).wait()

        for d in range(32):
            dv = jnp.full((16,), d, jnp.int32)
            for g in range(2):
                vals = plsc.load_gather(vtail, [dv, src[g]])
                plsc.store_scatter(vout0, [jl[g], dl[d]], vals)
        pltpu.async_copy(vout0.at[pl.ds(0, 8)],
                         te_hbm.at[pl.ds(_NCOL * 16, 8)], osem0)
        pltpu.make_async_copy(te_hbm.at[pl.ds(0, 8)], vout0.at[pl.ds(0, 8)],
                              osem0).wait()


@functools.partial(
    pl.kernel,
    mesh=_mesh,
    out_type=jax.ShapeDtypeStruct((_HIST, _DIM, _BATCH), jnp.float32),
    scratch_types=[
        pltpu.VMEM((_BLK,), jnp.int32),
        pltpu.VMEM((_BLK,), jnp.int32),
        pltpu.VMEM((2, 128), jnp.int32),
        pltpu.VMEM((2, 128), jnp.int32),
        pltpu.VMEM((_BLK,), jnp.int32),
        pltpu.VMEM((_BLK,), jnp.int32),
        pltpu.VMEM((_BLK, 128), jnp.float32),
        pltpu.VMEM((_BLK, 128), jnp.float32),
        pltpu.VMEM((_DIM, _BLK), jnp.float32),
        pltpu.VMEM((_DIM, _BLK), jnp.float32),
        pltpu.SemaphoreType.DMA,
        pltpu.SemaphoreType.DMA,
        pltpu.SemaphoreType.DMA,
        pltpu.SemaphoreType.DMA,
    ],
    compiler_params=pltpu.CompilerParams(needs_layout_passes=False),
)
def _gather(xt_hbm, te_hbm, out_hbm,
            ix0, ix1, ij0, ij1, xo0, xo1, rows0, rows1, dm0, dm1,
            gsem0, gsem1, osem0, osem1):
    wid = _wid()
    t0 = wid * _TPW
    it = _i16()
    li = [it + 16 * g for g in range(_BLK // 16)]

    ix = (ix0, ix1)
    ij = (ij0, ij1)
    xo = (xo0, xo1)
    rows = (rows0, rows1)
    dm = (dm0, dm1)
    gsem = (gsem0, gsem1)
    osem = (osem0, osem1)

    def task_hb(t):
        gt = t0 + t
        return gt // (_BATCH // _BLK), (gt % (_BATCH // _BLK)) * _BLK

    def prep(t, s):
        h, b0 = task_hb(t)
        pltpu.sync_copy(xt_hbm.at[h, pl.ds(b0, _BLK)], ix[s])

        def grp(k, c):
            v = ix[s][pl.ds(k * 16, 16)]
            ij[s][k // 8, pl.ds((k % 8) * 16, 16)] = v >> 2
            xo[s][pl.ds(k * 16, 16)] = (v & 3) * 32
            return c
        lax.fori_loop(0, _BLK // 16, grp, 0)
        for k in range(_BLK // 128):
            pltpu.async_copy(te_hbm.at[ij[s].at[k]],
                             rows[s].at[pl.ds(k * 128, 128)], gsem[s])

    def extract_and_out(t, s, first):
        h, b0 = task_hb(t)
        pltpu.make_async_copy(te_hbm.at[pl.ds(0, _BLK)], rows[s],
                              gsem[s]).wait()

        @pl.when(jnp.logical_not(first))
        def _():
            pltpu.make_async_copy(out_hbm.at[0, :, pl.ds(0, _BLK)], dm[s],
                                  osem[s]).wait()

        for g in range(_BLK // 16):
            xog = xo[s][pl.ds(16 * g, 16)]
            for d in range(32):
                vals = plsc.load_gather(rows[s], [li[g], xog + d])
                dm[s][d, pl.ds(16 * g, 16)] = vals
        pltpu.async_copy(dm[s], out_hbm.at[h, :, pl.ds(b0, _BLK)], osem[s])

    prep(0, 0)

    def pair(i2, carry):
        prep(2 * i2 + 1, 1)
        extract_and_out(2 * i2, 0, i2 == 0)

        @pl.when(i2 < _TPW // 2 - 1)
        def _():
            prep(2 * i2 + 2, 0)
        extract_and_out(2 * i2 + 1, 1, i2 == 0)
        return carry

    lax.fori_loop(0, _TPW // 2, pair, 0)
    for s in range(2):
        pltpu.make_async_copy(out_hbm.at[0, :, pl.ds(0, _BLK)], dm[s],
                              osem[s]).wait()


def kernel(x, table):
    xt = x.astype(jnp.int32).T
    tt = table.T
    te = _detrans(tt)
    out3 = _gather(xt, te)
    return out3.transpose(2, 0, 1)
